# Initial kernel scaffold; baseline (speedup 1.0000x reference)
import jax
import jax.numpy as jnp
from jax.experimental import pallas as pl


def kernel(x_question, x_answer, pos_edge_label_index, neg_edge_label_index):
    raise NotImplementedError("write your pallas kernel here")


# speedup vs baseline: 2.8889x; 2.8889x over previous
import functools

import jax
import jax.numpy as jnp
from jax import lax
from jax.experimental import pallas as pl
from jax.experimental.pallas import tpu as pltpu
from jax.experimental.pallas import tpu_sc as plsc

N = 10000
D = 128
E = 320000
NC = 2
NS = 16
NW = NC * NS
EPW = E // NW
W = 80
CHUNKS = EPW // W
NG = W // 16
SSTRIDE = 17


def _make_kernel():
    f32 = jnp.float32
    out_sd = jax.ShapeDtypeStruct((E,), f32)
    mesh = plsc.VectorSubcoreMesh(core_axis_name="c", subcore_axis_name="s")

    @functools.partial(
        pl.kernel,
        out_type=(out_sd, out_sd),
        mesh=mesh,
        compiler_params=pltpu.CompilerParams(needs_layout_passes=False),
        scratch_types=[
            pltpu.VMEM((CHUNKS, W), jnp.int32),
            pltpu.VMEM((CHUNKS, W), jnp.int32),
            pltpu.VMEM((W, D), f32),
            pltpu.VMEM((W, D), f32),
            pltpu.VMEM((EPW,), f32),
            pltpu.VMEM((16 * SSTRIDE,), f32),
            pltpu.SemaphoreType.DMA,
            pltpu.SemaphoreType.DMA,
        ],
    )
    def k(pqi, pai, nqi, nai, xq, xa, pos_out, neg_out,
          idxq_v, idxa_v, qrows, arows, outv, scr, sem_q, sem_a):
        wid = lax.axis_index("s") * NC + lax.axis_index("c")
        col = lax.iota(jnp.int32, 16) * SSTRIDE

        def do_set(idxq_hbm, idxa_hbm, out_hbm):
            pltpu.sync_copy(idxq_hbm.at[wid], idxq_v)
            pltpu.sync_copy(idxa_hbm.at[wid], idxa_v)

            @pl.loop(0, CHUNKS)
            def _(c):
                cq = pltpu.async_copy(xq.at[idxq_v.at[c]], qrows, sem_q)
                ca = pltpu.async_copy(xa.at[idxa_v.at[c]], arows, sem_a)
                cq.wait()
                ca.wait()
                for g in range(NG):
                    for e16 in range(16):
                        e = g * 16 + e16
                        acc = qrows[e, pl.ds(0, 16)] * arows[e, pl.ds(0, 16)]
                        for j in range(1, D // 16):
                            acc = acc + (qrows[e, pl.ds(16 * j, 16)]
                                         * arows[e, pl.ds(16 * j, 16)])
                        scr[pl.ds(e16 * SSTRIDE, 16)] = acc
                    dot = plsc.load_gather(scr, [col])
                    for l in range(1, 16):
                        dot = dot + plsc.load_gather(scr, [col + l])
                    outv[pl.ds(c * W + g * 16, 16)] = 1.0 / (1.0 + jnp.exp(-dot))

            pltpu.sync_copy(outv, out_hbm.at[pl.ds(wid * EPW, EPW)])

        do_set(pqi, pai, pos_out)
        do_set(nqi, nai, neg_out)

    return k


_edge_decoder = _make_kernel()


@jax.jit
def kernel(x_question, x_answer, pos_edge_label_index, neg_edge_label_index):
    pqi = pos_edge_label_index[0].reshape(NW, CHUNKS, W)
    pai = pos_edge_label_index[1].reshape(NW, CHUNKS, W)
    nqi = neg_edge_label_index[0].reshape(NW, CHUNKS, W)
    nai = neg_edge_label_index[1].reshape(NW, CHUNKS, W)
    pos_pred, neg_pred = _edge_decoder(pqi, pai, nqi, nai, x_question, x_answer)
    return pos_pred, neg_pred


# speedup vs baseline: 8.2278x; 2.8481x over previous
import functools

import jax
import jax.numpy as jnp
from jax import lax
from jax.experimental import pallas as pl
from jax.experimental.pallas import tpu as pltpu
from jax.experimental.pallas import tpu_sc as plsc

N = 10000
D = 128
E = 320000
NC = 2
NS = 16
NW = NC * NS
EPW = E // NW
W = 80
CHUNKS = EPW // W
PAIRS = (CHUNKS - 1) // 2
NG = W // 16
SSTRIDE = 17


def _make_kernel():
    f32 = jnp.float32
    out_sd = jax.ShapeDtypeStruct((E,), f32)
    mesh = plsc.VectorSubcoreMesh(core_axis_name="c", subcore_axis_name="s")

    @functools.partial(
        pl.kernel,
        out_type=(out_sd, out_sd),
        mesh=mesh,
        compiler_params=pltpu.CompilerParams(needs_layout_passes=False),
        scratch_types=[
            pltpu.VMEM((CHUNKS, W), jnp.int32),
            pltpu.VMEM((CHUNKS, W), jnp.int32),
            pltpu.VMEM((W, D), f32),
            pltpu.VMEM((W, D), f32),
            pltpu.VMEM((W, D), f32),
            pltpu.VMEM((W, D), f32),
            pltpu.VMEM((EPW,), f32),
            pltpu.VMEM((16 * SSTRIDE,), f32),
            pltpu.SemaphoreType.DMA,
            pltpu.SemaphoreType.DMA,
            pltpu.SemaphoreType.DMA,
            pltpu.SemaphoreType.DMA,
        ],
    )
    def k(pqi, pai, nqi, nai, xq, xa, pos_out, neg_out,
          idxq_v, idxa_v, qr0, ar0, qr1, ar1, outv, scr,
          sq0, sa0, sq1, sa1):
        wid = lax.axis_index("s") * NC + lax.axis_index("c")
        col = lax.iota(jnp.int32, 16) * SSTRIDE

        def do_set(idxq_hbm, idxa_hbm, out_hbm):
            pltpu.sync_copy(idxq_hbm.at[wid], idxq_v)
            pltpu.sync_copy(idxa_hbm.at[wid], idxa_v)

            def start(c, qr, ar, sq, sa):
                pltpu.async_copy(xq.at[idxq_v.at[c]], qr, sq)
                pltpu.async_copy(xa.at[idxa_v.at[c]], ar, sa)

            def wait(c, qr, ar, sq, sa):
                pltpu.make_async_copy(xq.at[idxq_v.at[c]], qr, sq).wait()
                pltpu.make_async_copy(xa.at[idxa_v.at[c]], ar, sa).wait()

            def compute(c, qr, ar):
                @pl.loop(0, NG)
                def _(g):
                    for e16 in range(16):
                        e = g * 16 + e16
                        acc = qr[e, pl.ds(0, 16)] * ar[e, pl.ds(0, 16)]
                        for j in range(1, D // 16):
                            acc = acc + (qr[e, pl.ds(16 * j, 16)]
                                         * ar[e, pl.ds(16 * j, 16)])
                        scr[pl.ds(e16 * SSTRIDE, 16)] = acc
                    dot = plsc.load_gather(scr, [col])
                    for l in range(1, 16):
                        dot = dot + plsc.load_gather(scr, [col + l])
                    outv[pl.ds(c * W + g * 16, 16)] = 1.0 / (1.0 + jnp.exp(-dot))


            start(0, qr0, ar0, sq0, sa0)
            start(1, qr1, ar1, sq1, sa1)

            @pl.loop(0, PAIRS)
            def _(c2):
                c = 2 * c2
                wait(c, qr0, ar0, sq0, sa0)
                compute(c, qr0, ar0)
                start(c + 2, qr0, ar0, sq0, sa0)
                wait(c + 1, qr1, ar1, sq1, sa1)
                compute(c + 1, qr1, ar1)

                @pl.when(c2 < PAIRS - 1)
                def _():
                    start(c + 3, qr1, ar1, sq1, sa1)

            wait(CHUNKS - 1, qr0, ar0, sq0, sa0)
            compute(CHUNKS - 1, qr0, ar0)

            pltpu.sync_copy(outv, out_hbm.at[pl.ds(wid * EPW, EPW)])

        do_set(pqi, pai, pos_out)
        do_set(nqi, nai, neg_out)

    return k


_edge_decoder = _make_kernel()


@jax.jit
def kernel(x_question, x_answer, pos_edge_label_index, neg_edge_label_index):
    pqi = pos_edge_label_index[0].reshape(NW, CHUNKS, W)
    pai = pos_edge_label_index[1].reshape(NW, CHUNKS, W)
    nqi = neg_edge_label_index[0].reshape(NW, CHUNKS, W)
    nai = neg_edge_label_index[1].reshape(NW, CHUNKS, W)
    pos_pred, neg_pred = _edge_decoder(pqi, pai, nqi, nai, x_question, x_answer)
    return pos_pred, neg_pred


# speedup vs baseline: 8.8945x; 1.0810x over previous
import functools

import jax
import jax.numpy as jnp
from jax import lax
from jax.experimental import pallas as pl
from jax.experimental.pallas import tpu as pltpu
from jax.experimental.pallas import tpu_sc as plsc

N = 10000
D = 128
E = 320000
NC = 2
NS = 16
NW = NC * NS
EPW = E // NW
W = 80
CHUNKS = EPW // W
PAIRS = (CHUNKS - 1) // 2
NG = W // 16
SSTRIDE = 17


def _make_kernel():
    f32 = jnp.float32
    i32 = jnp.int32
    out_sd = jax.ShapeDtypeStruct((E,), f32)
    mesh = plsc.VectorSubcoreMesh(core_axis_name="c", subcore_axis_name="s")

    @functools.partial(
        pl.kernel,
        out_type=(out_sd, out_sd),
        mesh=mesh,
        compiler_params=pltpu.CompilerParams(needs_layout_passes=False),
        scratch_types=[
            pltpu.VMEM_SHARED((N, D), f32),
            pltpu.VMEM((W,), i32),
            pltpu.VMEM((W,), i32),
            pltpu.VMEM((W,), i32),
            pltpu.VMEM((W,), i32),
            pltpu.VMEM((W, D), f32),
            pltpu.VMEM((W, D), f32),
            pltpu.VMEM((W, D), f32),
            pltpu.VMEM((W, D), f32),
            pltpu.VMEM((W,), f32),
            pltpu.VMEM((W,), f32),
            pltpu.VMEM((16 * SSTRIDE,), f32),
            pltpu.SemaphoreType.DMA,
            pltpu.SemaphoreType.DMA,
            pltpu.SemaphoreType.DMA,
            pltpu.SemaphoreType.DMA,
            pltpu.SemaphoreType.DMA,
            pltpu.SemaphoreType.DMA,
            pltpu.SemaphoreType.DMA,
            pltpu.SemaphoreType.DMA,
        ],
    )
    def k(pqi, pai, nqi, nai, xq, xa, pos_out, neg_out,
          sq, iq0, ia0, iq1, ia1, qr0, ar0, qr1, ar1, ov0, ov1, scr,
          is0, is1, qs0, as0, qs1, as1, os0, os1):
        cid = lax.axis_index("c")
        sid = lax.axis_index("s")
        wid = sid * NC + cid
        col = lax.iota(jnp.int32, 16) * SSTRIDE

        @pl.when(sid == 0)
        def _():
            pltpu.sync_copy(xq, sq)

        plsc.subcore_barrier()

        def do_set(idxq_hbm, idxa_hbm, out_hbm):
            base = wid * EPW

            def fetch_idx(c, iq, ia, isem):
                pltpu.async_copy(idxq_hbm.at[pl.ds(base + c * W, W)], iq, isem)
                pltpu.async_copy(idxa_hbm.at[pl.ds(base + c * W, W)], ia, isem)

            def wait_idx(c, iq, ia, isem):
                pltpu.make_async_copy(
                    idxq_hbm.at[pl.ds(base + c * W, W)], iq, isem).wait()
                pltpu.make_async_copy(
                    idxa_hbm.at[pl.ds(base + c * W, W)], ia, isem).wait()

            def start_gather(iq, ia, qr, ar, qsem, asem):
                pltpu.async_copy(sq.at[iq], qr, qsem)
                pltpu.async_copy(xa.at[ia], ar, asem)

            def wait_gather(iq, ia, qr, ar, qsem, asem):
                pltpu.make_async_copy(sq.at[iq], qr, qsem).wait()
                pltpu.make_async_copy(xa.at[ia], ar, asem).wait()

            def compute(qr, ar, ov):
                @pl.loop(0, NG)
                def _(g):
                    for e16 in range(16):
                        e = g * 16 + e16
                        acc = qr[e, pl.ds(0, 16)] * ar[e, pl.ds(0, 16)]
                        for j in range(1, D // 16):
                            acc = acc + (qr[e, pl.ds(16 * j, 16)]
                                         * ar[e, pl.ds(16 * j, 16)])
                        scr[pl.ds(e16 * SSTRIDE, 16)] = acc
                    dot = plsc.load_gather(scr, [col])
                    for l in range(1, 16):
                        dot = dot + plsc.load_gather(scr, [col + l])
                    ov[pl.ds(g * 16, 16)] = 1.0 / (1.0 + jnp.exp(-dot))

            def put_out(c, ov, osem):
                pltpu.async_copy(ov, out_hbm.at[pl.ds(base + c * W, W)], osem)

            def wait_out(c, ov, osem):
                pltpu.make_async_copy(
                    ov, out_hbm.at[pl.ds(base + c * W, W)], osem).wait()

            fetch_idx(0, iq0, ia0, is0)
            fetch_idx(1, iq1, ia1, is1)
            wait_idx(0, iq0, ia0, is0)
            start_gather(iq0, ia0, qr0, ar0, qs0, as0)
            wait_idx(1, iq1, ia1, is1)
            start_gather(iq1, ia1, qr1, ar1, qs1, as1)

            @pl.loop(0, PAIRS)
            def _(c2):
                c = 2 * c2

                wait_gather(iq0, ia0, qr0, ar0, qs0, as0)
                fetch_idx(c + 2, iq0, ia0, is0)

                @pl.when(c2 > 0)
                def _():
                    wait_out(c - 2, ov0, os0)

                compute(qr0, ar0, ov0)
                put_out(c, ov0, os0)
                wait_idx(c + 2, iq0, ia0, is0)
                start_gather(iq0, ia0, qr0, ar0, qs0, as0)

                wait_gather(iq1, ia1, qr1, ar1, qs1, as1)

                @pl.when(c2 < PAIRS - 1)
                def _():
                    fetch_idx(c + 3, iq1, ia1, is1)

                @pl.when(c2 > 0)
                def _():
                    wait_out(c - 1, ov1, os1)

                compute(qr1, ar1, ov1)
                put_out(c + 1, ov1, os1)

                @pl.when(c2 < PAIRS - 1)
                def _():
                    wait_idx(c + 3, iq1, ia1, is1)
                    start_gather(iq1, ia1, qr1, ar1, qs1, as1)

            wait_gather(iq0, ia0, qr0, ar0, qs0, as0)
            wait_out(CHUNKS - 3, ov0, os0)
            compute(qr0, ar0, ov0)
            put_out(CHUNKS - 1, ov0, os0)
            wait_out(CHUNKS - 2, ov1, os1)
            wait_out(CHUNKS - 1, ov0, os0)

        do_set(pqi, pai, pos_out)
        do_set(nqi, nai, neg_out)

    return k


_edge_decoder = _make_kernel()


@jax.jit
def kernel(x_question, x_answer, pos_edge_label_index, neg_edge_label_index):
    return _edge_decoder(
        pos_edge_label_index[0], pos_edge_label_index[1],
        neg_edge_label_index[0], neg_edge_label_index[1],
        x_question, x_answer)
